# Initial kernel scaffold; baseline (speedup 1.0000x reference)
import jax
import jax.numpy as jnp
from jax.experimental import pallas as pl


def kernel(x, edge_index, edge_label_index, weight1, weight2, skip_w0, skip_b0, msg_w0, msg_b0, skip_w1, skip_b1, msg_w1, msg_b1, complex_weight):
    raise NotImplementedError("write your pallas kernel here")


# speedup vs baseline: 21.7305x; 21.7305x over previous
import functools

import numpy as np
import jax
import jax.numpy as jnp
from jax import lax
from jax.experimental import pallas as pl
from jax.experimental.pallas import tpu as pltpu
from jax.experimental.pallas import tpu_sc as plsc

_N = 10000
_E = 320000
_D = 128
_P = 10000

_NC = 2
_NS = 16
_NW = _NC * _NS
_EW = _E // _NW
_CH = 400
_NCHUNK = _EW // _CH
_RW = _N // _NS

_HM = _N // 2
_HP = 5120
_BM = 1280
_BK = 1280


def _build_hilbert_circulants():
    g = np.fft.irfft(1j * np.ones(_N // 2 + 1), n=_N).astype(np.float32)
    i = np.arange(_HM, dtype=np.int64)
    d = i[:, None] - i[None, :]
    ce = g[(2 * d - 1) % _N]
    co = g[(2 * d + 1) % _N]
    cs = np.zeros((2, _HP, _HP), dtype=np.float32)
    cs[0, :_HM, :_HM] = ce
    cs[1, :_HM, :_HM] = co
    return cs


_CIRC = _build_hilbert_circulants()


_MESH = plsc.VectorSubcoreMesh(core_axis_name="c", subcore_axis_name="s")


@functools.partial(
    pl.kernel,
    mesh=_MESH,
    compiler_params=pltpu.CompilerParams(use_tc_tiling_on_sc=False),
    out_type=jax.ShapeDtypeStruct((_NC, _NS, _RW, 16), jnp.float32),
    scratch_types=[
        pltpu.VMEM((_CH,), jnp.int32),
        pltpu.VMEM((_CH, 16), jnp.float32),
        pltpu.VMEM_SHARED((_N, 16), jnp.float32),
    ],
)
def _sc_degree(src_hbm, zeros_hbm, ones_hbm, out_hbm, idx_v, ones_v, acc_sh):
    cid = lax.axis_index("c")
    sid = lax.axis_index("s")
    wid = sid * _NC + cid
    pltpu.sync_copy(zeros_hbm, acc_sh.at[pl.ds(sid * _RW, _RW)])
    pltpu.sync_copy(ones_hbm, ones_v)
    plsc.subcore_barrier()
    base = wid * _EW
    for c in range(_NCHUNK):
        pltpu.sync_copy(src_hbm.at[pl.ds(base + c * _CH, _CH)], idx_v)
        pltpu.sync_copy(ones_v, acc_sh.at[idx_v], add=True)
    plsc.subcore_barrier()
    pltpu.sync_copy(acc_sh.at[pl.ds(sid * _RW, _RW)], out_hbm.at[cid, sid])


_HD = _D // 2


@functools.partial(
    pl.kernel,
    mesh=_MESH,
    compiler_params=pltpu.CompilerParams(use_tc_tiling_on_sc=False),
    out_type=jax.ShapeDtypeStruct((2, _NC, _NS, _RW, _HD), jnp.float32),
    scratch_types=[
        pltpu.VMEM((_CH,), jnp.int32),
        pltpu.VMEM((_CH,), jnp.int32),
        pltpu.VMEM((_CH, _HD), jnp.float32),
        pltpu.VMEM_SHARED((_N, _HD), jnp.float32),
        pltpu.SemaphoreType.DMA,
    ],
)
def _sc_segment(table_lo, table_hi, src_hbm, dst_hbm, zrows_hbm, out_hbm,
                sidx_v, didx_v, rows_v, acc_sh, sem):
    cid = lax.axis_index("c")
    sid = lax.axis_index("s")
    wid = sid * _NC + cid
    base = wid * _EW
    for half, table_hbm in enumerate((table_lo, table_hi)):
        pltpu.sync_copy(zrows_hbm, acc_sh.at[pl.ds(sid * _RW, _RW)])
        plsc.subcore_barrier()
        for c in range(_NCHUNK):
            pltpu.sync_copy(src_hbm.at[pl.ds(base + c * _CH, _CH)], sidx_v)
            pltpu.sync_copy(dst_hbm.at[pl.ds(base + c * _CH, _CH)], didx_v)
            pltpu.async_copy(table_hbm.at[sidx_v], rows_v, sem).wait()
            pltpu.sync_copy(rows_v, acc_sh.at[didx_v], add=True)
        plsc.subcore_barrier()
        pltpu.sync_copy(acc_sh.at[pl.ds(sid * _RW, _RW)],
                        out_hbm.at[half, cid, sid])
        plsc.subcore_barrier()


_PW = 320


@functools.partial(
    pl.kernel,
    mesh=_MESH,
    compiler_params=pltpu.CompilerParams(use_tc_tiling_on_sc=False, needs_layout_passes=False),
    out_type=jax.ShapeDtypeStruct((_NW * _PW,), jnp.float32),
    scratch_types=[
        pltpu.VMEM((_N,), jnp.float32),
        pltpu.VMEM((_PW,), jnp.int32),
        pltpu.VMEM((_PW,), jnp.int32),
        pltpu.VMEM((_PW,), jnp.float32),
    ],
)
def _sc_decode(pred_hbm, i0_hbm, i1_hbm, out_hbm, pred_v, ia_v, ib_v, o_v):
    cid = lax.axis_index("c")
    sid = lax.axis_index("s")
    wid = sid * _NC + cid
    pltpu.sync_copy(pred_hbm, pred_v)
    pltpu.sync_copy(i0_hbm.at[pl.ds(wid * _PW, _PW)], ia_v)
    pltpu.sync_copy(i1_hbm.at[pl.ds(wid * _PW, _PW)], ib_v)
    for j in range(_PW // 16):
        sl = pl.ds(j * 16, 16)
        va = plsc.load_gather(pred_v, [ia_v[sl]])
        vb = plsc.load_gather(pred_v, [ib_v[sl]])
        o_v[sl] = va * vb
    pltpu.sync_copy(o_v, out_hbm.at[pl.ds(wid * _PW, _PW)])


_BMROW = 2000


def _prep_body(degp_ref, x_ref, fs_ref, dinvb_ref):
    deg = degp_ref[0][:, 0:1] + degp_ref[1][:, 0:1]
    dinv = jnp.where(deg > 0.0, lax.rsqrt(deg), 0.0)
    fs_ref[...] = x_ref[...] * dinv
    dinvb_ref[...] = jnp.broadcast_to(dinv, (_BMROW, _D))


def _tc_prep(degp, x):
    nb = _N // _BMROW
    return pl.pallas_call(
        _prep_body,
        grid=(nb,),
        in_specs=[
            pl.BlockSpec((2, _BMROW, 16), lambda m: (0, m, 0)),
            pl.BlockSpec((_BMROW, _D), lambda m: (m, 0)),
        ],
        out_specs=[
            pl.BlockSpec((_BMROW, _D), lambda m: (m, 0)),
            pl.BlockSpec((_BMROW, _D), lambda m: (m, 0)),
        ],
        out_shape=[
            jax.ShapeDtypeStruct((_N, _D), jnp.float32),
            jax.ShapeDtypeStruct((_N, _D), jnp.float32),
        ],
    )(degp, x)


def _layer_body(rp_ref, dinvb_ref, feats_ref, mwt_ref, swt_ref, bias_ref,
                h_ref, fsn_ref):
    agg = (rp_ref[0] + rp_ref[1]) * dinvb_ref[...]
    h = jnp.dot(agg, mwt_ref[...], preferred_element_type=jnp.float32)
    h = h + jnp.dot(feats_ref[...], swt_ref[...],
                    preferred_element_type=jnp.float32)
    h = h + bias_ref[...]
    h_ref[...] = h
    fsn_ref[...] = h * dinvb_ref[...]


def _tc_layer(rp, dinvb, feats, mwt, swt, bias):
    nb = _N // _BMROW
    return pl.pallas_call(
        _layer_body,
        grid=(nb,),
        in_specs=[
            pl.BlockSpec((2, _BMROW, _D), lambda m: (0, m, 0)),
            pl.BlockSpec((_BMROW, _D), lambda m: (m, 0)),
            pl.BlockSpec((_BMROW, _D), lambda m: (m, 0)),
            pl.BlockSpec((_D, _D), lambda m: (0, 0)),
            pl.BlockSpec((_D, _D), lambda m: (0, 0)),
            pl.BlockSpec((1, _D), lambda m: (0, 0)),
        ],
        out_specs=[
            pl.BlockSpec((_BMROW, _D), lambda m: (m, 0)),
            pl.BlockSpec((_BMROW, _D), lambda m: (m, 0)),
        ],
        out_shape=[
            jax.ShapeDtypeStruct((_N, _D), jnp.float32),
            jax.ShapeDtypeStruct((_N, _D), jnp.float32),
        ],
    )(rp, dinvb, feats, mwt, swt, bias)


def _hilbert_body(cs_ref, hopp_ref, hsame_ref, at_ref, bt_ref, w1t_ref,
                  w2c_ref, out_ref, acc_ref):
    k = pl.program_id(2)
    nk = pl.num_programs(2)

    @pl.when(k == 0)
    def _():
        acc_ref[...] = jnp.zeros((_BM, _D), jnp.float32)

    acc_ref[...] += jnp.dot(cs_ref[...], hopp_ref[...],
                            preferred_element_type=jnp.float32)

    @pl.when(k == nk - 1)
    def _():
        h2 = hsame_ref[...]
        h3 = h2 * (1.0 + at_ref[...]) + acc_ref[...] * bt_ref[...]
        nrm = jnp.maximum(
            jnp.sqrt(jnp.sum(h3 * h3, axis=1, keepdims=True)), 1e-12)
        hn = h3 / nrm
        t = jnp.maximum(
            jnp.dot(hn, w1t_ref[...], preferred_element_type=jnp.float32), 0.0)
        p = jnp.dot(t, w2c_ref[...], preferred_element_type=jnp.float32)
        out_ref[...] = jnp.broadcast_to(jax.nn.sigmoid(p), (_BM, _D))


def _tc_hilbert_mlp(cs, hopp, hsame, at, bt, w1t, w2c):
    mb = _HP // _BM
    kb = _HP // _BK
    return pl.pallas_call(
        _hilbert_body,
        grid=(2, mb, kb),
        in_specs=[
            pl.BlockSpec((None, _BM, _BK), lambda p, m, k: (p, m, k)),
            pl.BlockSpec((None, _BK, _D), lambda p, m, k: (p, k, 0)),
            pl.BlockSpec((None, _BM, _D), lambda p, m, k: (p, m, 0)),
            pl.BlockSpec((1, _D), lambda p, m, k: (0, 0)),
            pl.BlockSpec((1, _D), lambda p, m, k: (0, 0)),
            pl.BlockSpec((_D, _D), lambda p, m, k: (0, 0)),
            pl.BlockSpec((_D, 1), lambda p, m, k: (0, 0)),
        ],
        out_specs=pl.BlockSpec((None, _BM, _D), lambda p, m, k: (p, m, 0)),
        out_shape=jax.ShapeDtypeStruct((2, _HP, _D), jnp.float32),
        scratch_shapes=[pltpu.VMEM((_BM, _D), jnp.float32)],
    )(cs, hopp, hsame, at, bt, w1t, w2c)


def kernel(x, edge_index, edge_label_index, weight1, weight2,
           skip_w0, skip_b0, msg_w0, msg_b0,
           skip_w1, skip_b1, msg_w1, msg_b1, complex_weight):
    src = edge_index[0]
    dst = edge_index[1]

    zeros16 = jnp.zeros((_RW, 16), jnp.float32)
    ones16 = jnp.ones((_CH, 16), jnp.float32)
    zrows = jnp.zeros((_RW, _HD), jnp.float32)

    def _segment(feat):
        p = _sc_segment(feat[:, :_HD], feat[:, _HD:], src, dst, zrows)
        return p.transpose(1, 2, 3, 0, 4).reshape(2, _N, _D)

    degp = _sc_degree(src, zeros16, ones16).reshape(2, _N, 16)
    fs0, dinvb = _tc_prep(degp, x)

    rp0 = _segment(fs0)
    h1, fs1 = _tc_layer(rp0, dinvb, x, msg_w0.T, skip_w0.T,
                        (msg_b0 + skip_b0).reshape(1, _D))

    rp1 = _segment(fs1)
    h2, _ = _tc_layer(rp1, dinvb, h1, msg_w1.T, skip_w1.T,
                      (msg_b1 + skip_b1).reshape(1, _D))

    hr = h2.reshape(_HM, 2, _D)
    h_even = hr[:, 0]
    h_odd = hr[:, 1]
    pad = ((0, 0), (0, _HP - _HM), (0, 0))
    h_opp = jnp.pad(jnp.stack([h_odd, h_even]), pad)
    h_same = jnp.pad(jnp.stack([h_even, h_odd]), pad)

    at = complex_weight[:, 0].reshape(1, _D)
    bt = complex_weight[:, 1].reshape(1, _D)
    predp = _tc_hilbert_mlp(jnp.asarray(_CIRC), h_opp, h_same,
                            at, bt, weight1.T, weight2.T)
    pred = jnp.stack([predp[0, :_HM, 0], predp[1, :_HM, 0]],
                     axis=1).reshape(_N)

    i0 = jnp.pad(edge_label_index[0], (0, _NW * _PW - _P))
    i1 = jnp.pad(edge_label_index[1], (0, _NW * _PW - _P))
    out = _sc_decode(pred, i0, i1)
    return out[:_P]


# speedup vs baseline: 34.3868x; 1.5824x over previous
import functools

import numpy as np
import jax
import jax.numpy as jnp
from jax import lax
from jax.experimental import pallas as pl
from jax.experimental.pallas import tpu as pltpu
from jax.experimental.pallas import tpu_sc as plsc

_N = 10000
_E = 320000
_D = 128
_P = 10000

_NC = 2
_NS = 16
_NW = _NC * _NS
_EW = _E // _NW
_CH = 400
_NCHUNK = _EW // _CH
_RW = _N // _NS

_HM = _N // 2
_HP = 5120
_BM = 1280
_BK = 1280


def _build_hilbert_circulants():
    g = np.fft.irfft(1j * np.ones(_N // 2 + 1), n=_N).astype(np.float32)
    i = np.arange(_HM, dtype=np.int64)
    d = i[:, None] - i[None, :]
    ce = g[(2 * d - 1) % _N]
    co = g[(2 * d + 1) % _N]
    cs = np.zeros((2, _HP, _HP), dtype=np.float32)
    cs[0, :_HM, :_HM] = ce
    cs[1, :_HM, :_HM] = co
    return cs


_CIRC = _build_hilbert_circulants()


_MESH = plsc.VectorSubcoreMesh(core_axis_name="c", subcore_axis_name="s")


@functools.partial(
    pl.kernel,
    mesh=_MESH,
    compiler_params=pltpu.CompilerParams(use_tc_tiling_on_sc=False),
    out_type=jax.ShapeDtypeStruct((_NC, _N, 16), jnp.float32),
    scratch_types=[
        pltpu.VMEM((_NCHUNK, _CH), jnp.int32),
        pltpu.VMEM((_CH, 16), jnp.float32),
        pltpu.VMEM_SHARED((_N, 16), jnp.float32),
    ],
)
def _sc_degree(src2_hbm, zeros_hbm, ones_hbm, out_hbm, idx_v, ones_v, acc_sh):
    cid = lax.axis_index("c")
    sid = lax.axis_index("s")
    wid = sid * _NC + cid
    pltpu.sync_copy(zeros_hbm, acc_sh.at[pl.ds(sid * _RW, _RW)])
    pltpu.sync_copy(ones_hbm, ones_v)
    pltpu.sync_copy(src2_hbm.at[pl.ds(wid * _NCHUNK, _NCHUNK)], idx_v)
    plsc.subcore_barrier()
    for c in range(_NCHUNK):
        pltpu.sync_copy(ones_v, acc_sh.at[idx_v.at[c]], add=True)
    plsc.subcore_barrier()
    pltpu.sync_copy(acc_sh.at[pl.ds(sid * _RW, _RW)],
                    out_hbm.at[cid, pl.ds(sid * _RW, _RW)])


_HD = _D // 2


@functools.partial(
    pl.kernel,
    mesh=_MESH,
    compiler_params=pltpu.CompilerParams(use_tc_tiling_on_sc=False),
    out_type=jax.ShapeDtypeStruct((2, _NC, _N, _HD), jnp.float32),
    scratch_types=[
        pltpu.VMEM((_NCHUNK, _CH), jnp.int32),
        pltpu.VMEM((_NCHUNK, _CH), jnp.int32),
        pltpu.VMEM((_CH, _HD), jnp.float32),
        pltpu.VMEM((_CH, _HD), jnp.float32),
        pltpu.VMEM_SHARED((_N, _HD), jnp.float32),
        pltpu.SemaphoreType.DMA,
        pltpu.SemaphoreType.DMA,
    ],
)
def _sc_segment(table_lo, table_hi, src2_hbm, dst2_hbm, zrows_hbm, out_hbm,
                sidx_v, didx_v, rows0, rows1, acc_sh, sem0, sem1):
    cid = lax.axis_index("c")
    sid = lax.axis_index("s")
    wid = sid * _NC + cid
    rows = (rows0, rows1)
    sems = (sem0, sem1)
    pltpu.sync_copy(src2_hbm.at[pl.ds(wid * _NCHUNK, _NCHUNK)], sidx_v)
    pltpu.sync_copy(dst2_hbm.at[pl.ds(wid * _NCHUNK, _NCHUNK)], didx_v)
    for half, table_hbm in enumerate((table_lo, table_hi)):
        pltpu.sync_copy(zrows_hbm, acc_sh.at[pl.ds(sid * _RW, _RW)])
        plsc.subcore_barrier()
        pend = pltpu.async_copy(table_hbm.at[sidx_v.at[0]], rows[0], sems[0])
        for c in range(_NCHUNK):
            nxt = None
            if c + 1 < _NCHUNK:
                nxt = pltpu.async_copy(table_hbm.at[sidx_v.at[c + 1]],
                                       rows[(c + 1) % 2], sems[(c + 1) % 2])
            pend.wait()
            pltpu.sync_copy(rows[c % 2], acc_sh.at[didx_v.at[c]], add=True)
            pend = nxt
        plsc.subcore_barrier()
        pltpu.sync_copy(acc_sh.at[pl.ds(sid * _RW, _RW)],
                        out_hbm.at[half, cid, pl.ds(sid * _RW, _RW)])
        plsc.subcore_barrier()


_PW = 320


@functools.partial(
    pl.kernel,
    mesh=_MESH,
    compiler_params=pltpu.CompilerParams(use_tc_tiling_on_sc=False, needs_layout_passes=False),
    out_type=jax.ShapeDtypeStruct((_NW * _PW,), jnp.float32),
    scratch_types=[
        pltpu.VMEM((_N,), jnp.float32),
        pltpu.VMEM((_PW,), jnp.int32),
        pltpu.VMEM((_PW,), jnp.int32),
        pltpu.VMEM((_PW,), jnp.float32),
    ],
)
def _sc_decode(pred_hbm, i0_hbm, i1_hbm, out_hbm, pred_v, ia_v, ib_v, o_v):
    cid = lax.axis_index("c")
    sid = lax.axis_index("s")
    wid = sid * _NC + cid
    pltpu.sync_copy(pred_hbm, pred_v)
    pltpu.sync_copy(i0_hbm.at[pl.ds(wid * _PW, _PW)], ia_v)
    pltpu.sync_copy(i1_hbm.at[pl.ds(wid * _PW, _PW)], ib_v)
    for j in range(_PW // 16):
        sl = pl.ds(j * 16, 16)
        va = plsc.load_gather(pred_v, [ia_v[sl]])
        vb = plsc.load_gather(pred_v, [ib_v[sl]])
        o_v[sl] = va * vb
    pltpu.sync_copy(o_v, out_hbm.at[pl.ds(wid * _PW, _PW)])


_BMROW = 2000


def _prep_body(degp_ref, x_ref, fs_ref, dinvb_ref):
    deg = degp_ref[0][:, 0:1] + degp_ref[1][:, 0:1]
    dinv = jnp.where(deg > 0.0, lax.rsqrt(deg), 0.0)
    fs_ref[...] = x_ref[...] * dinv
    dinvb_ref[...] = jnp.broadcast_to(dinv, (_BMROW, _D))


def _tc_prep(degp, x):
    nb = _N // _BMROW
    return pl.pallas_call(
        _prep_body,
        grid=(nb,),
        in_specs=[
            pl.BlockSpec((2, _BMROW, 16), lambda m: (0, m, 0)),
            pl.BlockSpec((_BMROW, _D), lambda m: (m, 0)),
        ],
        out_specs=[
            pl.BlockSpec((_BMROW, _D), lambda m: (m, 0)),
            pl.BlockSpec((_BMROW, _D), lambda m: (m, 0)),
        ],
        out_shape=[
            jax.ShapeDtypeStruct((_N, _D), jnp.float32),
            jax.ShapeDtypeStruct((_N, _D), jnp.float32),
        ],
    )(degp, x)


def _layer_body(rlo_ref, rhi_ref, dinvb_ref, feats_ref, mwt_ref, swt_ref,
                bias_ref, h_ref, fsn_ref):
    agg = jnp.concatenate(
        [rlo_ref[0] + rlo_ref[1], rhi_ref[0] + rhi_ref[1]],
        axis=-1) * dinvb_ref[...]
    h = jnp.dot(agg, mwt_ref[...], preferred_element_type=jnp.float32)
    h = h + jnp.dot(feats_ref[...], swt_ref[...],
                    preferred_element_type=jnp.float32)
    h = h + bias_ref[...]
    h_ref[...] = h
    fsn_ref[...] = h * dinvb_ref[...]


def _tc_layer(rlo, rhi, dinvb, feats, mwt, swt, bias):
    nb = _N // _BMROW
    return pl.pallas_call(
        _layer_body,
        grid=(nb,),
        in_specs=[
            pl.BlockSpec((2, _BMROW, _HD), lambda m: (0, m, 0)),
            pl.BlockSpec((2, _BMROW, _HD), lambda m: (0, m, 0)),
            pl.BlockSpec((_BMROW, _D), lambda m: (m, 0)),
            pl.BlockSpec((_BMROW, _D), lambda m: (m, 0)),
            pl.BlockSpec((_D, _D), lambda m: (0, 0)),
            pl.BlockSpec((_D, _D), lambda m: (0, 0)),
            pl.BlockSpec((1, _D), lambda m: (0, 0)),
        ],
        out_specs=[
            pl.BlockSpec((_BMROW, _D), lambda m: (m, 0)),
            pl.BlockSpec((_BMROW, _D), lambda m: (m, 0)),
        ],
        out_shape=[
            jax.ShapeDtypeStruct((_N, _D), jnp.float32),
            jax.ShapeDtypeStruct((_N, _D), jnp.float32),
        ],
    )(rlo, rhi, dinvb, feats, mwt, swt, bias)


def _hilbert_body(cs_ref, hopp_ref, hsame_ref, at_ref, bt_ref, w1t_ref,
                  w2c_ref, out_ref, acc_ref):
    k = pl.program_id(2)
    nk = pl.num_programs(2)

    @pl.when(k == 0)
    def _():
        acc_ref[...] = jnp.zeros((_BM, _D), jnp.float32)

    acc_ref[...] += jnp.dot(cs_ref[...], hopp_ref[...],
                            preferred_element_type=jnp.float32)

    @pl.when(k == nk - 1)
    def _():
        h2 = hsame_ref[...]
        h3 = h2 * (1.0 + at_ref[...]) + acc_ref[...] * bt_ref[...]
        nrm = jnp.maximum(
            jnp.sqrt(jnp.sum(h3 * h3, axis=1, keepdims=True)), 1e-12)
        hn = h3 / nrm
        t = jnp.maximum(
            jnp.dot(hn, w1t_ref[...], preferred_element_type=jnp.float32), 0.0)
        p = jnp.dot(t, w2c_ref[...], preferred_element_type=jnp.float32)
        out_ref[...] = jnp.broadcast_to(jax.nn.sigmoid(p), (_BM, _D))


def _tc_hilbert_mlp(cs, hopp, hsame, at, bt, w1t, w2c):
    mb = _HP // _BM
    kb = _HP // _BK
    return pl.pallas_call(
        _hilbert_body,
        grid=(2, mb, kb),
        in_specs=[
            pl.BlockSpec((None, _BM, _BK), lambda p, m, k: (p, m, k)),
            pl.BlockSpec((None, _BK, _D), lambda p, m, k: (p, k, 0)),
            pl.BlockSpec((None, _BM, _D), lambda p, m, k: (p, m, 0)),
            pl.BlockSpec((1, _D), lambda p, m, k: (0, 0)),
            pl.BlockSpec((1, _D), lambda p, m, k: (0, 0)),
            pl.BlockSpec((_D, _D), lambda p, m, k: (0, 0)),
            pl.BlockSpec((_D, 1), lambda p, m, k: (0, 0)),
        ],
        out_specs=pl.BlockSpec((None, _BM, _D), lambda p, m, k: (p, m, 0)),
        out_shape=jax.ShapeDtypeStruct((2, _HP, _D), jnp.float32),
        scratch_shapes=[pltpu.VMEM((_BM, _D), jnp.float32)],
    )(cs, hopp, hsame, at, bt, w1t, w2c)


def kernel(x, edge_index, edge_label_index, weight1, weight2,
           skip_w0, skip_b0, msg_w0, msg_b0,
           skip_w1, skip_b1, msg_w1, msg_b1, complex_weight):
    src = edge_index[0]
    dst = edge_index[1]

    zeros16 = jnp.zeros((_RW, 16), jnp.float32)
    ones16 = jnp.ones((_CH, 16), jnp.float32)
    zrows = jnp.zeros((_RW, _HD), jnp.float32)

    src2 = src.reshape(_NW * _NCHUNK, _CH)
    dst2 = dst.reshape(_NW * _NCHUNK, _CH)

    def _segment(feat):
        return _sc_segment(feat[:, :_HD], feat[:, _HD:], src2, dst2, zrows)

    degp = _sc_degree(src2, zeros16, ones16)
    fs0, dinvb = _tc_prep(degp, x)

    rp0 = _segment(fs0)
    h1, fs1 = _tc_layer(rp0[0], rp0[1], dinvb, x, msg_w0.T, skip_w0.T,
                        (msg_b0 + skip_b0).reshape(1, _D))

    rp1 = _segment(fs1)
    h2, _ = _tc_layer(rp1[0], rp1[1], dinvb, h1, msg_w1.T, skip_w1.T,
                      (msg_b1 + skip_b1).reshape(1, _D))

    hr = h2.reshape(_HM, 2, _D)
    h_even = hr[:, 0]
    h_odd = hr[:, 1]
    pad = ((0, 0), (0, _HP - _HM), (0, 0))
    h_opp = jnp.pad(jnp.stack([h_odd, h_even]), pad)
    h_same = jnp.pad(jnp.stack([h_even, h_odd]), pad)

    at = complex_weight[:, 0].reshape(1, _D)
    bt = complex_weight[:, 1].reshape(1, _D)
    predp = _tc_hilbert_mlp(jnp.asarray(_CIRC), h_opp, h_same,
                            at, bt, weight1.T, weight2.T)
    pred = jnp.stack([predp[0, :_HM, 0], predp[1, :_HM, 0]],
                     axis=1).reshape(_N)

    i0 = jnp.pad(edge_label_index[0], (0, _NW * _PW - _P))
    i1 = jnp.pad(edge_label_index[1], (0, _NW * _PW - _P))
    out = _sc_decode(pred, i0, i1)
    return out[:_P]


# speedup vs baseline: 35.8642x; 1.0430x over previous
import functools

import numpy as np
import jax
import jax.numpy as jnp
from jax import lax
from jax.experimental import pallas as pl
from jax.experimental.pallas import tpu as pltpu
from jax.experimental.pallas import tpu_sc as plsc

_N = 10000
_E = 320000
_D = 128
_P = 10000

_NC = 2
_NS = 16
_NW = _NC * _NS
_EW = _E // _NW
_CH = 400
_NCHUNK = _EW // _CH
_RW = _N // _NS

_HM = _N // 2
_HP = 5120
_BM = 1280
_BK = 1280


def _build_hilbert_circulants():
    g = np.fft.irfft(1j * np.ones(_N // 2 + 1), n=_N).astype(np.float32)
    i = np.arange(_HM, dtype=np.int64)
    d = i[:, None] - i[None, :]
    ce = g[(2 * d - 1) % _N]
    co = g[(2 * d + 1) % _N]
    cs = np.zeros((2, _HP, _HP), dtype=np.float32)
    cs[0, :_HM, :_HM] = ce
    cs[1, :_HM, :_HM] = co
    return cs.astype(jnp.bfloat16)


_CIRC = _build_hilbert_circulants()


_MESH = plsc.VectorSubcoreMesh(core_axis_name="c", subcore_axis_name="s")


@functools.partial(
    pl.kernel,
    mesh=_MESH,
    compiler_params=pltpu.CompilerParams(use_tc_tiling_on_sc=False),
    out_type=jax.ShapeDtypeStruct((_NC, _N, 16), jnp.float32),
    scratch_types=[
        pltpu.VMEM((_NCHUNK, _CH), jnp.int32),
        pltpu.VMEM((_CH, 16), jnp.float32),
        pltpu.VMEM_SHARED((_N, 16), jnp.float32),
    ],
)
def _sc_degree(src2_hbm, zeros_hbm, ones_hbm, out_hbm, idx_v, ones_v, acc_sh):
    cid = lax.axis_index("c")
    sid = lax.axis_index("s")
    wid = sid * _NC + cid
    pltpu.sync_copy(zeros_hbm, acc_sh.at[pl.ds(sid * _RW, _RW)])
    pltpu.sync_copy(ones_hbm, ones_v)
    pltpu.sync_copy(src2_hbm.at[pl.ds(wid * _NCHUNK, _NCHUNK)], idx_v)
    plsc.subcore_barrier()
    for c in range(_NCHUNK):
        pltpu.sync_copy(ones_v, acc_sh.at[idx_v.at[c]], add=True)
    plsc.subcore_barrier()
    pltpu.sync_copy(acc_sh.at[pl.ds(sid * _RW, _RW)],
                    out_hbm.at[cid, pl.ds(sid * _RW, _RW)])


_HD = _D // 2


@functools.partial(
    pl.kernel,
    mesh=_MESH,
    compiler_params=pltpu.CompilerParams(use_tc_tiling_on_sc=False),
    out_type=jax.ShapeDtypeStruct((2, _NC, _N, _HD), jnp.float32),
    scratch_types=[
        pltpu.VMEM((_NCHUNK, _CH), jnp.int32),
        pltpu.VMEM((_NCHUNK, _CH), jnp.int32),
        pltpu.VMEM((_CH, _HD), jnp.float32),
        pltpu.VMEM((_CH, _HD), jnp.float32),
        pltpu.VMEM_SHARED((_N, _HD), jnp.float32),
        pltpu.SemaphoreType.DMA,
        pltpu.SemaphoreType.DMA,
    ],
)
def _sc_segment(table_lo, table_hi, src2_hbm, dst2_hbm, zrows_hbm, out_hbm,
                sidx_v, didx_v, rows0, rows1, acc_sh, sem0, sem1):
    cid = lax.axis_index("c")
    sid = lax.axis_index("s")
    wid = sid * _NC + cid
    rows = (rows0, rows1)
    sems = (sem0, sem1)
    pltpu.sync_copy(src2_hbm.at[pl.ds(wid * _NCHUNK, _NCHUNK)], sidx_v)
    pltpu.sync_copy(dst2_hbm.at[pl.ds(wid * _NCHUNK, _NCHUNK)], didx_v)
    for half, table_hbm in enumerate((table_lo, table_hi)):
        pltpu.sync_copy(zrows_hbm, acc_sh.at[pl.ds(sid * _RW, _RW)])
        plsc.subcore_barrier()
        pend = pltpu.async_copy(table_hbm.at[sidx_v.at[0]], rows[0], sems[0])
        for c in range(_NCHUNK):
            nxt = None
            if c + 1 < _NCHUNK:
                nxt = pltpu.async_copy(table_hbm.at[sidx_v.at[c + 1]],
                                       rows[(c + 1) % 2], sems[(c + 1) % 2])
            pend.wait()
            pltpu.sync_copy(rows[c % 2], acc_sh.at[didx_v.at[c]], add=True)
            pend = nxt
        plsc.subcore_barrier()
        pltpu.sync_copy(acc_sh.at[pl.ds(sid * _RW, _RW)],
                        out_hbm.at[half, cid, pl.ds(sid * _RW, _RW)])
        plsc.subcore_barrier()


_PW = 320


@functools.partial(
    pl.kernel,
    mesh=_MESH,
    compiler_params=pltpu.CompilerParams(use_tc_tiling_on_sc=False, needs_layout_passes=False),
    out_type=jax.ShapeDtypeStruct((_NW * _PW,), jnp.float32),
    scratch_types=[
        pltpu.VMEM((_N,), jnp.float32),
        pltpu.VMEM((_PW,), jnp.int32),
        pltpu.VMEM((_PW,), jnp.int32),
        pltpu.VMEM((_PW,), jnp.float32),
    ],
)
def _sc_decode(pred_hbm, i0_hbm, i1_hbm, out_hbm, pred_v, ia_v, ib_v, o_v):
    cid = lax.axis_index("c")
    sid = lax.axis_index("s")
    wid = sid * _NC + cid
    pltpu.sync_copy(pred_hbm, pred_v)
    pltpu.sync_copy(i0_hbm.at[pl.ds(wid * _PW, _PW)], ia_v)
    pltpu.sync_copy(i1_hbm.at[pl.ds(wid * _PW, _PW)], ib_v)
    for j in range(_PW // 16):
        sl = pl.ds(j * 16, 16)
        va = plsc.load_gather(pred_v, [ia_v[sl]])
        vb = plsc.load_gather(pred_v, [ib_v[sl]])
        o_v[sl] = va * vb
    pltpu.sync_copy(o_v, out_hbm.at[pl.ds(wid * _PW, _PW)])


_BMROW = 2000


def _prep_body(degp_ref, x_ref, fs_ref, dinvb_ref):
    deg = degp_ref[0][:, 0:1] + degp_ref[1][:, 0:1]
    dinv = jnp.where(deg > 0.0, lax.rsqrt(deg), 0.0)
    fs_ref[...] = x_ref[...] * dinv
    dinvb_ref[...] = jnp.broadcast_to(dinv, (_BMROW, _D))


def _tc_prep(degp, x):
    nb = _N // _BMROW
    return pl.pallas_call(
        _prep_body,
        grid=(nb,),
        in_specs=[
            pl.BlockSpec((2, _BMROW, 16), lambda m: (0, m, 0)),
            pl.BlockSpec((_BMROW, _D), lambda m: (m, 0)),
        ],
        out_specs=[
            pl.BlockSpec((_BMROW, _D), lambda m: (m, 0)),
            pl.BlockSpec((_BMROW, _D), lambda m: (m, 0)),
        ],
        out_shape=[
            jax.ShapeDtypeStruct((_N, _D), jnp.float32),
            jax.ShapeDtypeStruct((_N, _D), jnp.float32),
        ],
    )(degp, x)


def _layer_body(rlo_ref, rhi_ref, dinvb_ref, feats_ref, mwt_ref, swt_ref,
                bias_ref, h_ref, fsn_ref):
    agg = jnp.concatenate(
        [rlo_ref[0] + rlo_ref[1], rhi_ref[0] + rhi_ref[1]],
        axis=-1) * dinvb_ref[...]
    h = jnp.dot(agg, mwt_ref[...], preferred_element_type=jnp.float32)
    h = h + jnp.dot(feats_ref[...], swt_ref[...],
                    preferred_element_type=jnp.float32)
    h = h + bias_ref[...]
    h_ref[...] = h
    fsn_ref[...] = h * dinvb_ref[...]


def _tc_layer(rlo, rhi, dinvb, feats, mwt, swt, bias):
    nb = _N // _BMROW
    return pl.pallas_call(
        _layer_body,
        grid=(nb,),
        in_specs=[
            pl.BlockSpec((2, _BMROW, _HD), lambda m: (0, m, 0)),
            pl.BlockSpec((2, _BMROW, _HD), lambda m: (0, m, 0)),
            pl.BlockSpec((_BMROW, _D), lambda m: (m, 0)),
            pl.BlockSpec((_BMROW, _D), lambda m: (m, 0)),
            pl.BlockSpec((_D, _D), lambda m: (0, 0)),
            pl.BlockSpec((_D, _D), lambda m: (0, 0)),
            pl.BlockSpec((1, _D), lambda m: (0, 0)),
        ],
        out_specs=[
            pl.BlockSpec((_BMROW, _D), lambda m: (m, 0)),
            pl.BlockSpec((_BMROW, _D), lambda m: (m, 0)),
        ],
        out_shape=[
            jax.ShapeDtypeStruct((_N, _D), jnp.float32),
            jax.ShapeDtypeStruct((_N, _D), jnp.float32),
        ],
    )(rlo, rhi, dinvb, feats, mwt, swt, bias)


def _hilbert_body(cs_ref, hopp_ref, hsame_ref, at_ref, bt_ref, w1t_ref,
                  w2c_ref, out_ref, acc_ref):
    k = pl.program_id(2)
    nk = pl.num_programs(2)

    @pl.when(k == 0)
    def _():
        acc_ref[...] = jnp.zeros((_BM, _D), jnp.float32)

    acc_ref[...] += jnp.dot(cs_ref[...], hopp_ref[...].astype(jnp.bfloat16),
                            preferred_element_type=jnp.float32)

    @pl.when(k == nk - 1)
    def _():
        h2 = hsame_ref[...]
        h3 = h2 * (1.0 + at_ref[...]) + acc_ref[...] * bt_ref[...]
        nrm = jnp.maximum(
            jnp.sqrt(jnp.sum(h3 * h3, axis=1, keepdims=True)), 1e-12)
        hn = h3 / nrm
        t = jnp.maximum(
            jnp.dot(hn, w1t_ref[...], preferred_element_type=jnp.float32), 0.0)
        p = jnp.dot(t, w2c_ref[...], preferred_element_type=jnp.float32)
        out_ref[...] = jnp.broadcast_to(jax.nn.sigmoid(p), (_BM, _D))


def _tc_hilbert_mlp(cs, hopp, hsame, at, bt, w1t, w2c):
    mb = _HP // _BM
    kb = _HP // _BK
    return pl.pallas_call(
        _hilbert_body,
        grid=(2, mb, kb),
        in_specs=[
            pl.BlockSpec((None, _BM, _BK), lambda p, m, k: (p, m, k)),
            pl.BlockSpec((None, _BK, _D), lambda p, m, k: (p, k, 0)),
            pl.BlockSpec((None, _BM, _D), lambda p, m, k: (p, m, 0)),
            pl.BlockSpec((1, _D), lambda p, m, k: (0, 0)),
            pl.BlockSpec((1, _D), lambda p, m, k: (0, 0)),
            pl.BlockSpec((_D, _D), lambda p, m, k: (0, 0)),
            pl.BlockSpec((_D, 1), lambda p, m, k: (0, 0)),
        ],
        out_specs=pl.BlockSpec((None, _BM, _D), lambda p, m, k: (p, m, 0)),
        out_shape=jax.ShapeDtypeStruct((2, _HP, _D), jnp.float32),
        scratch_shapes=[pltpu.VMEM((_BM, _D), jnp.float32)],
    )(cs, hopp, hsame, at, bt, w1t, w2c)


def kernel(x, edge_index, edge_label_index, weight1, weight2,
           skip_w0, skip_b0, msg_w0, msg_b0,
           skip_w1, skip_b1, msg_w1, msg_b1, complex_weight):
    src = edge_index[0]
    dst = edge_index[1]

    zeros16 = jnp.zeros((_RW, 16), jnp.float32)
    ones16 = jnp.ones((_CH, 16), jnp.float32)
    zrows = jnp.zeros((_RW, _HD), jnp.float32)

    src2 = src.reshape(_NW * _NCHUNK, _CH)
    dst2 = dst.reshape(_NW * _NCHUNK, _CH)

    def _segment(feat):
        return _sc_segment(feat[:, :_HD], feat[:, _HD:], src2, dst2, zrows)

    degp = _sc_degree(src2, zeros16, ones16)
    fs0, dinvb = _tc_prep(degp, x)

    rp0 = _segment(fs0)
    h1, fs1 = _tc_layer(rp0[0], rp0[1], dinvb, x, msg_w0.T, skip_w0.T,
                        (msg_b0 + skip_b0).reshape(1, _D))

    rp1 = _segment(fs1)
    h2, _ = _tc_layer(rp1[0], rp1[1], dinvb, h1, msg_w1.T, skip_w1.T,
                      (msg_b1 + skip_b1).reshape(1, _D))

    hr = h2.reshape(_HM, 2, _D)
    h_even = hr[:, 0]
    h_odd = hr[:, 1]
    pad = ((0, 0), (0, _HP - _HM), (0, 0))
    h_opp = jnp.pad(jnp.stack([h_odd, h_even]), pad)
    h_same = jnp.pad(jnp.stack([h_even, h_odd]), pad)

    at = complex_weight[:, 0].reshape(1, _D)
    bt = complex_weight[:, 1].reshape(1, _D)
    predp = _tc_hilbert_mlp(jnp.asarray(_CIRC), h_opp, h_same,
                            at, bt, weight1.T, weight2.T)
    pred = jnp.stack([predp[0, :_HM, 0], predp[1, :_HM, 0]],
                     axis=1).reshape(_N)

    i0 = jnp.pad(edge_label_index[0], (0, _NW * _PW - _P))
    i1 = jnp.pad(edge_label_index[1], (0, _NW * _PW - _P))
    out = _sc_decode(pred, i0, i1)
    return out[:_P]


# speedup vs baseline: 36.2151x; 1.0098x over previous
import functools

import numpy as np
import jax
import jax.numpy as jnp
from jax import lax
from jax.experimental import pallas as pl
from jax.experimental.pallas import tpu as pltpu
from jax.experimental.pallas import tpu_sc as plsc

_N = 10000
_E = 320000
_D = 128
_P = 10000

_NC = 2
_NS = 16
_NW = _NC * _NS
_EW = _E // _NW
_CH = 400
_NCHUNK = _EW // _CH
_RW = _N // _NS

_HM = _N // 2
_HP = 5120
_BM = 1280
_BK = 1280


def _build_hilbert_circulants():
    g = np.fft.irfft(1j * np.ones(_N // 2 + 1), n=_N).astype(np.float32)
    i = np.arange(_HM, dtype=np.int64)
    d = i[:, None] - i[None, :]
    ce = g[(2 * d - 1) % _N]
    co = g[(2 * d + 1) % _N]
    cs = np.zeros((2, _HP, _HP), dtype=np.float32)
    cs[0, :_HM, :_HM] = ce
    cs[1, :_HM, :_HM] = co
    return cs.astype(jnp.bfloat16)


_CIRC = _build_hilbert_circulants()


_MESH = plsc.VectorSubcoreMesh(core_axis_name="c", subcore_axis_name="s")


@functools.partial(
    pl.kernel,
    mesh=_MESH,
    compiler_params=pltpu.CompilerParams(use_tc_tiling_on_sc=False),
    out_type=jax.ShapeDtypeStruct((_NC, _N, 16), jnp.float32),
    scratch_types=[
        pltpu.VMEM((_NCHUNK, _CH), jnp.int32),
        pltpu.VMEM((_CH, 16), jnp.float32),
        pltpu.VMEM_SHARED((_N, 16), jnp.float32),
    ],
)
def _sc_degree(src2_hbm, zeros_hbm, ones_hbm, out_hbm, idx_v, ones_v, acc_sh):
    cid = lax.axis_index("c")
    sid = lax.axis_index("s")
    wid = sid * _NC + cid
    pltpu.sync_copy(zeros_hbm, acc_sh.at[pl.ds(sid * _RW, _RW)])
    pltpu.sync_copy(ones_hbm, ones_v)
    pltpu.sync_copy(src2_hbm.at[pl.ds(wid * _NCHUNK, _NCHUNK)], idx_v)
    plsc.subcore_barrier()
    for c in range(_NCHUNK):
        pltpu.sync_copy(ones_v, acc_sh.at[idx_v.at[c]], add=True)
    plsc.subcore_barrier()
    pltpu.sync_copy(acc_sh.at[pl.ds(sid * _RW, _RW)],
                    out_hbm.at[cid, pl.ds(sid * _RW, _RW)])


_HD = _D // 2


@functools.partial(
    pl.kernel,
    mesh=_MESH,
    compiler_params=pltpu.CompilerParams(use_tc_tiling_on_sc=False),
    out_type=jax.ShapeDtypeStruct((2, _NC, _N, _HD), jnp.float32),
    scratch_types=[
        pltpu.VMEM((_NCHUNK, _CH), jnp.int32),
        pltpu.VMEM((_NCHUNK, _CH), jnp.int32),
        pltpu.VMEM((_CH, _HD), jnp.float32),
        pltpu.VMEM((_CH, _HD), jnp.float32),
        pltpu.VMEM_SHARED((_N, _HD), jnp.float32),
        pltpu.SemaphoreType.DMA,
        pltpu.SemaphoreType.DMA,
    ],
)
def _sc_segment(table_lo, table_hi, src2_hbm, dst2_hbm, zrows_hbm, out_hbm,
                sidx_v, didx_v, rows0, rows1, acc_sh, sem0, sem1):
    cid = lax.axis_index("c")
    sid = lax.axis_index("s")
    wid = sid * _NC + cid
    rows = (rows0, rows1)
    sems = (sem0, sem1)
    nbuf = 2
    pltpu.sync_copy(src2_hbm.at[pl.ds(wid * _NCHUNK, _NCHUNK)], sidx_v)
    pltpu.sync_copy(dst2_hbm.at[pl.ds(wid * _NCHUNK, _NCHUNK)], didx_v)
    for half, table_hbm in enumerate((table_lo, table_hi)):
        pltpu.sync_copy(zrows_hbm, acc_sh.at[pl.ds(sid * _RW, _RW)])
        plsc.subcore_barrier()
        pend = [pltpu.async_copy(table_hbm.at[sidx_v.at[c]],
                                 rows[c % nbuf], sems[c % nbuf])
                for c in range(nbuf - 1)]
        for c in range(_NCHUNK):
            nc = c + nbuf - 1
            if nc < _NCHUNK:
                pend.append(pltpu.async_copy(table_hbm.at[sidx_v.at[nc]],
                                             rows[nc % nbuf], sems[nc % nbuf]))
            pend.pop(0).wait()
            pltpu.sync_copy(rows[c % nbuf], acc_sh.at[didx_v.at[c]], add=True)
        plsc.subcore_barrier()
        pltpu.sync_copy(acc_sh.at[pl.ds(sid * _RW, _RW)],
                        out_hbm.at[half, cid, pl.ds(sid * _RW, _RW)])
        plsc.subcore_barrier()


_PW = 320


@functools.partial(
    pl.kernel,
    mesh=_MESH,
    compiler_params=pltpu.CompilerParams(use_tc_tiling_on_sc=False, needs_layout_passes=False),
    out_type=jax.ShapeDtypeStruct((_NW * _PW,), jnp.float32),
    scratch_types=[
        pltpu.VMEM((_N,), jnp.float32),
        pltpu.VMEM((_PW,), jnp.int32),
        pltpu.VMEM((_PW,), jnp.int32),
        pltpu.VMEM((_PW,), jnp.float32),
    ],
)
def _sc_decode(pred_hbm, i0_hbm, i1_hbm, out_hbm, pred_v, ia_v, ib_v, o_v):
    cid = lax.axis_index("c")
    sid = lax.axis_index("s")
    wid = sid * _NC + cid
    pltpu.sync_copy(pred_hbm, pred_v)
    pltpu.sync_copy(i0_hbm.at[pl.ds(wid * _PW, _PW)], ia_v)
    pltpu.sync_copy(i1_hbm.at[pl.ds(wid * _PW, _PW)], ib_v)
    for j in range(_PW // 16):
        sl = pl.ds(j * 16, 16)
        va = plsc.load_gather(pred_v, [ia_v[sl]])
        vb = plsc.load_gather(pred_v, [ib_v[sl]])
        o_v[sl] = va * vb
    pltpu.sync_copy(o_v, out_hbm.at[pl.ds(wid * _PW, _PW)])


_BMROW = 2000


def _prep_body(degp_ref, x_ref, fslo_ref, fshi_ref, dinvb_ref):
    deg = degp_ref[0][:, 0:1] + degp_ref[1][:, 0:1]
    dinv = jnp.where(deg > 0.0, lax.rsqrt(deg), 0.0)
    fs = x_ref[...] * dinv
    fslo_ref[...] = fs[:, :_HD]
    fshi_ref[...] = fs[:, _HD:]
    dinvb_ref[...] = jnp.broadcast_to(dinv, (_BMROW, _D))


def _tc_prep(degp, x):
    nb = _N // _BMROW
    return pl.pallas_call(
        _prep_body,
        grid=(nb,),
        in_specs=[
            pl.BlockSpec((2, _BMROW, 16), lambda m: (0, m, 0)),
            pl.BlockSpec((_BMROW, _D), lambda m: (m, 0)),
        ],
        out_specs=[
            pl.BlockSpec((_BMROW, _HD), lambda m: (m, 0)),
            pl.BlockSpec((_BMROW, _HD), lambda m: (m, 0)),
            pl.BlockSpec((_BMROW, _D), lambda m: (m, 0)),
        ],
        out_shape=[
            jax.ShapeDtypeStruct((_N, _HD), jnp.float32),
            jax.ShapeDtypeStruct((_N, _HD), jnp.float32),
            jax.ShapeDtypeStruct((_N, _D), jnp.float32),
        ],
    )(degp, x)


def _layer_body(rlo_ref, rhi_ref, dinvb_ref, feats_ref, mwt_ref, swt_ref,
                bias_ref, h_ref, fsnlo_ref, fsnhi_ref):
    agg = jnp.concatenate(
        [rlo_ref[0] + rlo_ref[1], rhi_ref[0] + rhi_ref[1]],
        axis=-1) * dinvb_ref[...]
    h = jnp.dot(agg, mwt_ref[...], preferred_element_type=jnp.float32)
    h = h + jnp.dot(feats_ref[...], swt_ref[...],
                    preferred_element_type=jnp.float32)
    h = h + bias_ref[...]
    h_ref[...] = h
    fsn = h * dinvb_ref[...]
    fsnlo_ref[...] = fsn[:, :_HD]
    fsnhi_ref[...] = fsn[:, _HD:]


def _tc_layer(rlo, rhi, dinvb, feats, mwt, swt, bias):
    nb = _N // _BMROW
    return pl.pallas_call(
        _layer_body,
        grid=(nb,),
        in_specs=[
            pl.BlockSpec((2, _BMROW, _HD), lambda m: (0, m, 0)),
            pl.BlockSpec((2, _BMROW, _HD), lambda m: (0, m, 0)),
            pl.BlockSpec((_BMROW, _D), lambda m: (m, 0)),
            pl.BlockSpec((_BMROW, _D), lambda m: (m, 0)),
            pl.BlockSpec((_D, _D), lambda m: (0, 0)),
            pl.BlockSpec((_D, _D), lambda m: (0, 0)),
            pl.BlockSpec((1, _D), lambda m: (0, 0)),
        ],
        out_specs=[
            pl.BlockSpec((_BMROW, _D), lambda m: (m, 0)),
            pl.BlockSpec((_BMROW, _HD), lambda m: (m, 0)),
            pl.BlockSpec((_BMROW, _HD), lambda m: (m, 0)),
        ],
        out_shape=[
            jax.ShapeDtypeStruct((_N, _D), jnp.float32),
            jax.ShapeDtypeStruct((_N, _HD), jnp.float32),
            jax.ShapeDtypeStruct((_N, _HD), jnp.float32),
        ],
    )(rlo, rhi, dinvb, feats, mwt, swt, bias)


def _hilbert_body(cs_ref, hopp_ref, hsame_ref, at_ref, bt_ref, w1t_ref,
                  w2c_ref, out_ref, acc_ref):
    k = pl.program_id(2)
    nk = pl.num_programs(2)

    @pl.when(k == 0)
    def _():
        acc_ref[...] = jnp.zeros((_BM, _D), jnp.float32)

    acc_ref[...] += jnp.dot(cs_ref[...], hopp_ref[...].astype(jnp.bfloat16),
                            preferred_element_type=jnp.float32)

    @pl.when(k == nk - 1)
    def _():
        h2 = hsame_ref[...]
        h3 = h2 * (1.0 + at_ref[...]) + acc_ref[...] * bt_ref[...]
        nrm = jnp.maximum(
            jnp.sqrt(jnp.sum(h3 * h3, axis=1, keepdims=True)), 1e-12)
        hn = h3 / nrm
        t = jnp.maximum(
            jnp.dot(hn, w1t_ref[...], preferred_element_type=jnp.float32), 0.0)
        p = jnp.dot(t, w2c_ref[...], preferred_element_type=jnp.float32)
        out_ref[...] = jnp.broadcast_to(jax.nn.sigmoid(p), (_BM, _D))


def _tc_hilbert_mlp(cs, hopp, hsame, at, bt, w1t, w2c):
    mb = _HP // _BM
    kb = _HP // _BK
    return pl.pallas_call(
        _hilbert_body,
        grid=(2, mb, kb),
        in_specs=[
            pl.BlockSpec((None, _BM, _BK), lambda p, m, k: (p, m, k)),
            pl.BlockSpec((None, _BK, _D), lambda p, m, k: (p, k, 0)),
            pl.BlockSpec((None, _BM, _D), lambda p, m, k: (p, m, 0)),
            pl.BlockSpec((1, _D), lambda p, m, k: (0, 0)),
            pl.BlockSpec((1, _D), lambda p, m, k: (0, 0)),
            pl.BlockSpec((_D, _D), lambda p, m, k: (0, 0)),
            pl.BlockSpec((_D, 1), lambda p, m, k: (0, 0)),
        ],
        out_specs=pl.BlockSpec((None, _BM, _D), lambda p, m, k: (p, m, 0)),
        out_shape=jax.ShapeDtypeStruct((2, _HP, _D), jnp.float32),
        scratch_shapes=[pltpu.VMEM((_BM, _D), jnp.float32)],
    )(cs, hopp, hsame, at, bt, w1t, w2c)


def kernel(x, edge_index, edge_label_index, weight1, weight2,
           skip_w0, skip_b0, msg_w0, msg_b0,
           skip_w1, skip_b1, msg_w1, msg_b1, complex_weight):
    src = edge_index[0]
    dst = edge_index[1]

    zeros16 = jnp.zeros((_RW, 16), jnp.float32)
    ones16 = jnp.ones((_CH, 16), jnp.float32)
    zrows = jnp.zeros((_RW, _HD), jnp.float32)

    src2 = src.reshape(_NW * _NCHUNK, _CH)
    dst2 = dst.reshape(_NW * _NCHUNK, _CH)

    degp = _sc_degree(src2, zeros16, ones16)
    fs0lo, fs0hi, dinvb = _tc_prep(degp, x)

    rp0 = _sc_segment(fs0lo, fs0hi, src2, dst2, zrows)
    h1, fs1lo, fs1hi = _tc_layer(rp0[0], rp0[1], dinvb, x, msg_w0.T,
                                 skip_w0.T, (msg_b0 + skip_b0).reshape(1, _D))

    rp1 = _sc_segment(fs1lo, fs1hi, src2, dst2, zrows)
    h2, _, _ = _tc_layer(rp1[0], rp1[1], dinvb, h1, msg_w1.T, skip_w1.T,
                         (msg_b1 + skip_b1).reshape(1, _D))

    hr = h2.reshape(_HM, 2, _D)
    h_even = hr[:, 0]
    h_odd = hr[:, 1]
    pad = ((0, 0), (0, _HP - _HM), (0, 0))
    h_opp = jnp.pad(jnp.stack([h_odd, h_even]), pad)
    h_same = jnp.pad(jnp.stack([h_even, h_odd]), pad)

    at = complex_weight[:, 0].reshape(1, _D)
    bt = complex_weight[:, 1].reshape(1, _D)
    predp = _tc_hilbert_mlp(jnp.asarray(_CIRC), h_opp, h_same,
                            at, bt, weight1.T, weight2.T)
    pred = jnp.stack([predp[0, :_HM, 0], predp[1, :_HM, 0]],
                     axis=1).reshape(_N)

    i0 = jnp.pad(edge_label_index[0], (0, _NW * _PW - _P))
    i1 = jnp.pad(edge_label_index[1], (0, _NW * _PW - _P))
    out = _sc_decode(pred, i0, i1)
    return out[:_P]
